# Initial kernel scaffold; baseline (speedup 1.0000x reference)
import jax
import jax.numpy as jnp
from jax.experimental import pallas as pl


def kernel(target_tensor, support_tensor, weight, bias):
    raise NotImplementedError("write your pallas kernel here")


# speedup vs baseline: 1.3714x; 1.3714x over previous
import functools

import jax
import jax.numpy as jnp
from jax.experimental import pallas as pl
from jax.experimental.pallas import tpu as pltpu


def _cross_op_body(tgt_ref, sup_ref, wt_ref, ws_ref, bias_ref,
                   agg_ref, inter_ref, ct_ref, *, S, C, Co, H, W):
    HW = H * W
    s = pl.program_id(1)

    col = jax.lax.broadcasted_iota(jnp.int32, (C, HW), 1) & (W - 1)
    not_first_col = col != 0
    not_last_col = col != (W - 1)
    lane = jax.lax.broadcasted_iota(jnp.int32, (Co, HW), 1)
    not_first_row = lane >= W
    not_last_row = lane < (HW - W)

    def conv3(x, wref):
        xs_m = jnp.where(not_first_col, jnp.roll(x, 1, axis=1), 0.0)
        xs_p = jnp.where(not_last_col, jnp.roll(x, -1, axis=1), 0.0)
        x3 = jnp.concatenate([xs_m, x, xs_p], axis=0)
        p0 = jnp.dot(wref[0], x3, preferred_element_type=jnp.float32)
        p1 = jnp.dot(wref[1], x3, preferred_element_type=jnp.float32)
        p2 = jnp.dot(wref[2], x3, preferred_element_type=jnp.float32)
        up = jnp.where(not_first_row, jnp.roll(p0, W, axis=1), 0.0)
        dn = jnp.where(not_last_row, jnp.roll(p2, -W, axis=1), 0.0)
        return p1 + up + dn

    @pl.when(s == 0)
    def _():
        ct_ref[...] = conv3(tgt_ref[0], wt_ref) + bias_ref[...]

    out = conv3(sup_ref[0, 0], ws_ref) + ct_ref[...]
    inter_ref[0, 0] = out

    @pl.when(s == 0)
    def _():
        agg_ref[0, 0] = out

    @pl.when(s != 0)
    def _():
        agg_ref[0, 0] = agg_ref[0, 0] + out

    @pl.when(s == S - 1)
    def _():
        agg_ref[0, 0] = agg_ref[0, 0] * (1.0 / S)


def kernel(target_tensor, support_tensor, weight, bias):
    B, T, C, H, W = target_tensor.shape
    S = support_tensor.shape[1]
    Co = weight.shape[0]
    HW = H * W

    tgt = target_tensor.reshape(B, C, HW)
    sup = support_tensor.reshape(B, S, C, HW)
    w_t = jnp.transpose(weight[:, :C], (2, 0, 3, 1)).reshape(3, Co, 3 * C)
    w_s = jnp.transpose(weight[:, C:], (2, 0, 3, 1)).reshape(3, Co, 3 * C)
    bias2 = bias.reshape(Co, 1)

    agg, inter = pl.pallas_call(
        functools.partial(_cross_op_body, S=S, C=C, Co=Co, H=H, W=W),
        grid=(B, S),
        in_specs=[
            pl.BlockSpec((1, C, HW), lambda b, s: (b, 0, 0)),
            pl.BlockSpec((1, 1, C, HW), lambda b, s: (b, s, 0, 0)),
            pl.BlockSpec((3, Co, 3 * C), lambda b, s: (0, 0, 0)),
            pl.BlockSpec((3, Co, 3 * C), lambda b, s: (0, 0, 0)),
            pl.BlockSpec((Co, 1), lambda b, s: (0, 0)),
        ],
        out_specs=[
            pl.BlockSpec((1, 1, Co, HW), lambda b, s: (b, 0, 0, 0)),
            pl.BlockSpec((1, 1, Co, HW), lambda b, s: (b, s, 0, 0)),
        ],
        out_shape=[
            jax.ShapeDtypeStruct((B, 1, Co, HW), jnp.float32),
            jax.ShapeDtypeStruct((B, S, Co, HW), jnp.float32),
        ],
        scratch_shapes=[pltpu.VMEM((Co, HW), jnp.float32)],
        compiler_params=pltpu.CompilerParams(
            dimension_semantics=("parallel", "arbitrary"),
        ),
        name="cross_op_fused",
    )(tgt, sup, w_t, w_s, bias2)

    aggregated = agg.reshape(B, 1, Co, H, W)
    interactions = inter.reshape(B, S, Co, H, W)
    return aggregated, interactions


# speedup vs baseline: 1.3944x; 1.0168x over previous
import functools

import jax
import jax.numpy as jnp
from jax.experimental import pallas as pl
from jax.experimental.pallas import tpu as pltpu


def _cross_op_body(tgt_ref, sup_ref, wt_ref, ws_ref, bias_ref,
                   agg_ref, inter_ref, ct_ref, *, S, C, Co, H, W):
    HW = H * W
    s = pl.program_id(1)

    col = jax.lax.broadcasted_iota(jnp.int32, (C, HW), 1) & (W - 1)
    not_first_col = col != 0
    not_last_col = col != (W - 1)
    lane = jax.lax.broadcasted_iota(jnp.int32, (Co, HW), 1)
    not_first_row = lane >= W
    not_last_row = lane < (HW - W)

    def conv3(x, wref):
        xs_m = jnp.where(not_first_col, jnp.roll(x, 1, axis=1), 0.0)
        xs_p = jnp.where(not_last_col, jnp.roll(x, -1, axis=1), 0.0)
        x3 = jnp.concatenate([xs_m, x, xs_p], axis=0).astype(jnp.bfloat16)
        p = jnp.dot(wref[...], x3, preferred_element_type=jnp.float32)
        p0, p1, p2 = p[:Co], p[Co:2 * Co], p[2 * Co:]
        up = jnp.where(not_first_row, jnp.roll(p0, W, axis=1), 0.0)
        dn = jnp.where(not_last_row, jnp.roll(p2, -W, axis=1), 0.0)
        return p1 + up + dn

    @pl.when(s == 0)
    def _():
        ct_ref[...] = conv3(tgt_ref[0], wt_ref) + bias_ref[...]

    out = conv3(sup_ref[0, 0], ws_ref) + ct_ref[...]
    inter_ref[0, 0] = out

    @pl.when(s == 0)
    def _():
        agg_ref[0, 0] = out

    @pl.when(s != 0)
    def _():
        agg_ref[0, 0] = agg_ref[0, 0] + out

    @pl.when(s == S - 1)
    def _():
        agg_ref[0, 0] = agg_ref[0, 0] * (1.0 / S)


def kernel(target_tensor, support_tensor, weight, bias):
    B, T, C, H, W = target_tensor.shape
    S = support_tensor.shape[1]
    Co = weight.shape[0]
    HW = H * W

    tgt = target_tensor.reshape(B, C, HW)
    sup = support_tensor.reshape(B, S, C, HW)
    w_t = (jnp.transpose(weight[:, :C], (2, 0, 3, 1))
           .reshape(3 * Co, 3 * C).astype(jnp.bfloat16))
    w_s = (jnp.transpose(weight[:, C:], (2, 0, 3, 1))
           .reshape(3 * Co, 3 * C).astype(jnp.bfloat16))
    bias2 = bias.reshape(Co, 1)

    agg, inter = pl.pallas_call(
        functools.partial(_cross_op_body, S=S, C=C, Co=Co, H=H, W=W),
        grid=(B, S),
        in_specs=[
            pl.BlockSpec((1, C, HW), lambda b, s: (b, 0, 0)),
            pl.BlockSpec((1, 1, C, HW), lambda b, s: (b, s, 0, 0)),
            pl.BlockSpec((3 * Co, 3 * C), lambda b, s: (0, 0)),
            pl.BlockSpec((3 * Co, 3 * C), lambda b, s: (0, 0)),
            pl.BlockSpec((Co, 1), lambda b, s: (0, 0)),
        ],
        out_specs=[
            pl.BlockSpec((1, 1, Co, HW), lambda b, s: (b, 0, 0, 0)),
            pl.BlockSpec((1, 1, Co, HW), lambda b, s: (b, s, 0, 0)),
        ],
        out_shape=[
            jax.ShapeDtypeStruct((B, 1, Co, HW), jnp.float32),
            jax.ShapeDtypeStruct((B, S, Co, HW), jnp.float32),
        ],
        scratch_shapes=[pltpu.VMEM((Co, HW), jnp.float32)],
        compiler_params=pltpu.CompilerParams(
            dimension_semantics=("parallel", "arbitrary"),
        ),
        name="cross_op_fused",
    )(tgt, sup, w_t, w_s, bias2)

    aggregated = agg.reshape(B, 1, Co, H, W)
    interactions = inter.reshape(B, S, Co, H, W)
    return aggregated, interactions


# speedup vs baseline: 1.4042x; 1.0070x over previous
import functools

import jax
import jax.numpy as jnp
from jax.experimental import pallas as pl
from jax.experimental.pallas import tpu as pltpu


def _cross_op_body(tgt_ref, sup_ref, wt_ref, ws_ref, bias_ref,
                   agg_ref, inter_ref, ct_ref, *, S, G, C, Co, H, W):
    HW = H * W
    sg = pl.program_id(1)

    col = jax.lax.broadcasted_iota(jnp.int32, (C, HW), 1) & (W - 1)
    not_first_col = col != 0
    not_last_col = col != (W - 1)
    lane = jax.lax.broadcasted_iota(jnp.int32, (Co, HW), 1)
    not_first_row = lane >= W
    not_last_row = lane < (HW - W)

    def conv3(x, wref):
        xs_m = jnp.where(not_first_col, jnp.roll(x, 1, axis=1), 0.0)
        xs_p = jnp.where(not_last_col, jnp.roll(x, -1, axis=1), 0.0)
        x3 = jnp.concatenate([xs_m, x, xs_p], axis=0).astype(jnp.bfloat16)
        p = jnp.dot(wref[...], x3, preferred_element_type=jnp.float32)
        p0, p1, p2 = p[:Co], p[Co:2 * Co], p[2 * Co:]
        up = jnp.where(not_first_row, jnp.roll(p0, W, axis=1), 0.0)
        dn = jnp.where(not_last_row, jnp.roll(p2, -W, axis=1), 0.0)
        return p1 + up + dn

    @pl.when(sg == 0)
    def _():
        ct_ref[...] = conv3(tgt_ref[0], wt_ref) + bias_ref[...]

    ct = ct_ref[...]
    acc = None
    for g in range(G):
        out = conv3(sup_ref[0, g], ws_ref) + ct
        inter_ref[0, g] = out
        acc = out if acc is None else acc + out

    @pl.when(sg == 0)
    def _():
        agg_ref[0, 0] = acc

    @pl.when(sg != 0)
    def _():
        agg_ref[0, 0] = agg_ref[0, 0] + acc

    @pl.when(sg == S // G - 1)
    def _():
        agg_ref[0, 0] = agg_ref[0, 0] * (1.0 / S)


def kernel(target_tensor, support_tensor, weight, bias):
    B, T, C, H, W = target_tensor.shape
    S = support_tensor.shape[1]
    Co = weight.shape[0]
    HW = H * W
    G = 4

    tgt = target_tensor.reshape(B, C, HW)
    sup = support_tensor.reshape(B, S, C, HW)
    w_t = (jnp.transpose(weight[:, :C], (2, 0, 3, 1))
           .reshape(3 * Co, 3 * C).astype(jnp.bfloat16))
    w_s = (jnp.transpose(weight[:, C:], (2, 0, 3, 1))
           .reshape(3 * Co, 3 * C).astype(jnp.bfloat16))
    bias2 = bias.reshape(Co, 1)

    agg, inter = pl.pallas_call(
        functools.partial(_cross_op_body, S=S, G=G, C=C, Co=Co, H=H, W=W),
        grid=(B, S // G),
        in_specs=[
            pl.BlockSpec((1, C, HW), lambda b, s: (b, 0, 0)),
            pl.BlockSpec((1, G, C, HW), lambda b, s: (b, s, 0, 0)),
            pl.BlockSpec((3 * Co, 3 * C), lambda b, s: (0, 0)),
            pl.BlockSpec((3 * Co, 3 * C), lambda b, s: (0, 0)),
            pl.BlockSpec((Co, 1), lambda b, s: (0, 0)),
        ],
        out_specs=[
            pl.BlockSpec((1, 1, Co, HW), lambda b, s: (b, 0, 0, 0)),
            pl.BlockSpec((1, G, Co, HW), lambda b, s: (b, s, 0, 0)),
        ],
        out_shape=[
            jax.ShapeDtypeStruct((B, 1, Co, HW), jnp.float32),
            jax.ShapeDtypeStruct((B, S, Co, HW), jnp.float32),
        ],
        scratch_shapes=[pltpu.VMEM((Co, HW), jnp.float32)],
        compiler_params=pltpu.CompilerParams(
            dimension_semantics=("parallel", "arbitrary"),
            vmem_limit_bytes=64 * 1024 * 1024,
        ),
        name="cross_op_fused",
    )(tgt, sup, w_t, w_s, bias2)

    aggregated = agg.reshape(B, 1, Co, H, W)
    interactions = inter.reshape(B, S, Co, H, W)
    return aggregated, interactions
